# Initial kernel scaffold; baseline (speedup 1.0000x reference)
import jax
import jax.numpy as jnp
from jax.experimental import pallas as pl


def kernel(x):
    raise NotImplementedError("write your pallas kernel here")


# speedup vs baseline: 24.6848x; 24.6848x over previous
import jax
import jax.numpy as jnp
from jax import lax
from jax.experimental import pallas as pl
from jax.experimental.pallas import tpu as pltpu

CONF_T = 0.25
IOU_T = 0.45
MAX_WH = 4096.0
M = 1024
NDET = 300
NPAD = 320
N = 20000
NP = 20480
NR = 160
B = 8
CH = 2048
NCH = NP // CH
HI = lax.Precision.HIGHEST
F32 = jnp.float32
I32 = jnp.int32
SDS = jax.ShapeDtypeStruct


def _f2key(s):
    bits = lax.bitcast_convert_type(s, I32)
    return jnp.where(bits >= 0, bits, jnp.bitwise_xor(bits, jnp.int32(0x7FFFFFFF)))


def _excl_cumsum(m):
    x = jnp.where(m, 1.0, 0.0)
    l = lax.broadcasted_iota(I32, (128, 128), 0)
    c = lax.broadcasted_iota(I32, (128, 128), 1)
    upper = jnp.where(l < c, 1.0, 0.0)
    lane_ex = lax.dot_general(x, upper, (((1,), (0,)), ((), ())), precision=HI)
    rt = jnp.sum(x, axis=1, keepdims=True)
    r = lax.broadcasted_iota(I32, (NR, NR), 0)
    cc = lax.broadcasted_iota(I32, (NR, NR), 1)
    lower = jnp.where(cc < r, 1.0, 0.0)
    row_ex = lax.dot_general(lower, rt, (((1,), (0,)), ((), ())), precision=HI)
    return lane_ex + row_ex


def _pack_matrix():
    j = lax.broadcasted_iota(I32, (M, 64), 0)
    w = lax.broadcasted_iota(I32, (M, 64), 1)
    pw = jnp.left_shift(jnp.int32(1), jnp.bitwise_and(j, 15))
    return jnp.where(lax.shift_right_logical(j, 4) == w, pw, 0).astype(F32)


def _score_kernel(x_ref, v_ref, s_ref):
    v = x_ref[0]
    obj = v[:, 4:5]
    p = v[:, 5:85] * obj
    conf = jnp.max(p, axis=1, keepdims=True)
    io = lax.broadcasted_iota(I32, (2000, 80), 1)
    cls = jnp.min(jnp.where(p == conf, io, 80), axis=1, keepdims=True)
    valid = (obj > CONF_T) & (conf > CONF_T)
    score = jnp.where(valid, conf, -1.0)
    xy = v[:, 0:2]
    half = v[:, 2:4] * 0.5
    v_ref[0] = jnp.concatenate(
        [xy - half, xy + half, score, cls.astype(F32),
         jnp.zeros((2000, 2), F32)], axis=1)
    s_ref[0] = score


_score_call = pl.pallas_call(
    _score_kernel,
    grid=(B, 10),
    in_specs=[pl.BlockSpec((1, 2000, 85), lambda i, c: (i, c, 0))],
    out_specs=[pl.BlockSpec((1, 2000, 8), lambda i, c: (i, c, 0)),
               pl.BlockSpec((1, 2000, 1), lambda i, c: (i, c, 0))],
    out_shape=[SDS((B, N, 8), F32), SDS((B, N, 1), F32)],
)


def _thresh_kernel(s_ref, t_ref):
    keys = _f2key(s_ref[...])

    def body(bi, t):
        trial = t + jnp.left_shift(jnp.int32(1), 30 - bi)
        cnt = jnp.sum(jnp.where(keys >= trial, 1.0, 0.0), axis=(1, 2),
                      keepdims=True)
        return jnp.where(cnt >= float(M), trial, t)

    cnt0 = jnp.sum(jnp.where(keys >= 0, 1.0, 0.0), axis=(1, 2), keepdims=True)
    t0 = jnp.where(cnt0 >= float(M), jnp.int32(0),
                   jnp.full((B, 1, 1), jnp.iinfo(jnp.int32).min, I32))
    t = lax.fori_loop(0, 31, body, t0)
    c_above = jnp.sum(jnp.where(keys > t, 1.0, 0.0), axis=(1, 2),
                      keepdims=True).astype(I32)
    io = lax.broadcasted_iota(I32, (B, 1, 128), 2)
    t_ref[...] = jnp.where(io == 0, t, jnp.where(io == 1, c_above, 0))


_thresh_call = pl.pallas_call(
    _thresh_kernel,
    in_specs=[pl.BlockSpec((B, NR, 128), lambda: (0, 0, 0))],
    out_specs=pl.BlockSpec((B, 1, 128), lambda: (0, 0, 0)),
    out_shape=SDS((B, 1, 128), I32),
)


def _compact_kernel(s_ref, v_ref, t_ref, so_ref, st_ref, pos_s, sel_s):
    s = s_ref[0]
    keys = _f2key(s)
    t = t_ref[0]
    thr = t[0:1, 0:1]
    c_above = t[0:1, 1:2].astype(F32)
    m_gt = keys > thr
    m_eq = keys == thr
    ex_eq = _excl_cumsum(m_eq)
    n_need = float(M) - c_above
    sel = m_gt | (m_eq & (ex_eq < n_need))
    pos = _excl_cumsum(sel)
    pos_s[...] = pos.reshape(NCH, 1, CH)
    sel_s[...] = jnp.where(sel, 1.0, 0.0).reshape(NCH, 1, CH)

    iom = lax.broadcasted_iota(I32, (M, CH), 0).astype(F32)

    def chunk_body(c, acc):
        pc = pos_s[c]
        sc = sel_s[c]
        mc = jnp.where((iom == pc) & (sc > 0.5), 1.0, 0.0)
        vc = v_ref[0, pl.ds(c * CH, CH), :]
        return acc + lax.dot_general(mc, vc, (((1,), (0,)), ((), ())),
                                     precision=HI)

    acc = lax.fori_loop(0, NCH, chunk_body, jnp.zeros((M, 8), F32))

    acc_t = jnp.transpose(acc, (1, 0))
    kcol = _f2key(acc[:, 4:5])
    krow = _f2key(acc_t[4:5, :])
    icol = lax.broadcasted_iota(I32, (M, M), 0)
    irow = lax.broadcasted_iota(I32, (M, M), 1)
    beats = (krow > kcol) | ((krow == kcol) & (irow < icol))
    rank = lax.dot_general(jnp.where(beats, 1.0, 0.0), jnp.ones((M, 1), F32),
                           (((1,), (0,)), ((), ())), precision=HI)
    rank_row = jnp.transpose(rank, (1, 0))
    iop = lax.broadcasted_iota(I32, (M, M), 0).astype(F32)
    perm = jnp.where(iop == rank_row, 1.0, 0.0)
    srt = lax.dot_general(perm, acc, (((1,), (0,)), ((), ())), precision=HI)
    so_ref[0] = srt
    st_ref[0] = jnp.transpose(srt, (1, 0))


_compact_call = pl.pallas_call(
    _compact_kernel,
    grid=(B,),
    in_specs=[pl.BlockSpec((1, NR, 128), lambda i: (i, 0, 0)),
              pl.BlockSpec((1, NP, 8), lambda i: (i, 0, 0)),
              pl.BlockSpec((1, 1, 128), lambda i: (i, 0, 0))],
    out_specs=[pl.BlockSpec((1, M, 8), lambda i: (i, 0, 0)),
               pl.BlockSpec((1, 8, M), lambda i: (i, 0, 0))],
    out_shape=[SDS((B, M, 8), F32), SDS((B, 8, M), F32)],
    scratch_shapes=[pltpu.VMEM((NCH, 1, CH), F32),
                    pltpu.VMEM((NCH, 1, CH), F32)],
)


def _iou_kernel(so_ref, st_ref, ap_ref, vp_ref):
    bc = so_ref[0]
    br = st_ref[0]
    offc = bc[:, 5:6] * MAX_WH
    offr = br[5:6, :] * MAX_WH
    x1 = bc[:, 0:1] + offc
    y1 = bc[:, 1:2] + offc
    x2 = bc[:, 2:3] + offc
    y2 = bc[:, 3:4] + offc
    xx1 = br[0:1, :] + offr
    yy1 = br[1:2, :] + offr
    xx2 = br[2:3, :] + offr
    yy2 = br[3:4, :] + offr
    w = jnp.clip(jnp.minimum(x2, xx2) - jnp.maximum(x1, xx1), 0.0, None)
    h = jnp.clip(jnp.minimum(y2, yy2) - jnp.maximum(y1, yy1), 0.0, None)
    inter = w * h
    area = (x2 - x1) * (y2 - y1)
    area_t = (xx2 - xx1) * (yy2 - yy1)
    iou = inter / (area + area_t - inter + 1e-9)
    adj = jnp.where(iou > IOU_T, 1.0, 0.0)
    wp = _pack_matrix()
    ap_ref[0] = lax.dot_general(adj, wp, (((1,), (0,)), ((), ())),
                                precision=HI)
    vmask = jnp.where(br[4:5, :] > 0.0, 1.0, 0.0)
    vp_ref[0] = lax.dot_general(vmask, wp, (((1,), (0,)), ((), ())),
                                precision=HI)


_iou_call = pl.pallas_call(
    _iou_kernel,
    grid=(B,),
    in_specs=[pl.BlockSpec((1, M, 8), lambda i: (i, 0, 0)),
              pl.BlockSpec((1, 8, M), lambda i: (i, 0, 0))],
    out_specs=[pl.BlockSpec((1, M, 64), lambda i: (i, 0, 0)),
               pl.BlockSpec((1, 1, 64), lambda i: (i, 0, 0))],
    out_shape=[SDS((B, M, 64), F32), SDS((B, 1, 64), F32)],
)


def _greedy_kernel(ap_ref, vp_ref, keep_ref):
    validb = vp_ref[...][:, 0, :].astype(I32)
    io64 = lax.broadcasted_iota(I32, (1, 64), 1)
    io1k = lax.broadcasted_iota(I32, (B, M), 1)

    def body(i, carry):
        kbits, kf = carry
        row = ap_ref[i].astype(I32)
        supp = jnp.max(jnp.where((row & kbits) != 0, 1, 0), axis=1,
                       keepdims=True)
        word = lax.shift_right_logical(i, 4)
        bit = jnp.left_shift(jnp.int32(1), jnp.bitwise_and(i, 15))
        mvec = jnp.where(io64 == word, bit, 0)
        vbit = jnp.max(jnp.where((validb & mvec) != 0, 1, 0), axis=1,
                       keepdims=True)
        ok = (supp == 0) & (vbit == 1)
        kbits = kbits | jnp.where(ok, mvec, 0)
        kf = kf + jnp.where((io1k == i) & ok, 1.0, 0.0)
        return kbits, kf

    _, kf = lax.fori_loop(
        0, M, body,
        (jnp.zeros((B, 64), I32), jnp.zeros((B, M), F32)))
    keep_ref[...] = kf


_greedy_call = pl.pallas_call(
    _greedy_kernel,
    in_specs=[pl.BlockSpec((M, B, 64), lambda: (0, 0, 0)),
              pl.BlockSpec((B, 1, 64), lambda: (0, 0, 0))],
    out_specs=pl.BlockSpec((B, M), lambda: (0, 0)),
    out_shape=SDS((B, M), F32),
)


def _final_kernel(k_ref, so_ref, o_ref):
    i = pl.program_id(0)
    krow = k_ref[pl.ds(i, 1), :]
    kcol = jnp.transpose(krow, (1, 0))
    r = lax.broadcasted_iota(I32, (M, M), 0)
    c = lax.broadcasted_iota(I32, (M, M), 1)
    lower = jnp.where(c < r, 1.0, 0.0)
    pos = lax.dot_general(lower, kcol, (((1,), (0,)), ((), ())), precision=HI)
    pos_row = jnp.transpose(pos, (1, 0))
    iop = lax.broadcasted_iota(I32, (NPAD, M), 0).astype(F32)
    p2 = jnp.where((iop == pos_row) & (krow > 0.5), 1.0, 0.0)
    o_ref[0] = lax.dot_general(p2, so_ref[0], (((1,), (0,)), ((), ())),
                               precision=HI)


_final_call = pl.pallas_call(
    _final_kernel,
    grid=(B,),
    in_specs=[pl.BlockSpec((B, M), lambda i: (0, 0)),
              pl.BlockSpec((1, M, 8), lambda i: (i, 0, 0))],
    out_specs=pl.BlockSpec((1, NPAD, 8), lambda i: (i, 0, 0)),
    out_shape=SDS((B, NPAD, 8), F32),
)


def kernel(x):
    pred = x[0]
    v, s = _score_call(pred)
    s_pad = jnp.concatenate(
        [s[..., 0], jnp.full((B, NP - N), -1.0, F32)], axis=1)
    s3 = s_pad.reshape(B, NR, 128)
    v_pad = jnp.concatenate([v, jnp.zeros((B, NP - N, 8), F32)], axis=1)
    tinfo = _thresh_call(s3)
    srt, srt_t = _compact_call(s3, v_pad, tinfo)
    adjp, validp = _iou_call(srt, srt_t)
    keep = _greedy_call(jnp.transpose(adjp, (1, 0, 2)), validp)
    det = _final_call(keep, srt)
    return det[:, :NDET, :6]


# speedup vs baseline: 29.1932x; 1.1826x over previous
import jax
import jax.numpy as jnp
from jax import lax
from jax.experimental import pallas as pl
from jax.experimental.pallas import tpu as pltpu

CONF_T = 0.25
IOU_T = 0.45
MAX_WH = 4096.0
M = 1024
NDET = 300
NPAD = 320
N = 20000
NP = 20480
NR = 160
B = 8
CH = 2048
NCH = NP // CH
HI = lax.Precision.HIGHEST
F32 = jnp.float32
I32 = jnp.int32
SDS = jax.ShapeDtypeStruct
KEYNEG = -1065353217


def _f2key(s):
    bits = lax.bitcast_convert_type(s, I32)
    return jnp.where(bits >= 0, bits, jnp.bitwise_xor(bits, jnp.int32(0x7FFFFFFF)))


def _excl_cumsum(m):
    x = jnp.where(m, 1.0, 0.0)
    l = lax.broadcasted_iota(I32, (128, 128), 0)
    c = lax.broadcasted_iota(I32, (128, 128), 1)
    upper = jnp.where(l < c, 1.0, 0.0)
    lane_ex = lax.dot_general(x, upper, (((1,), (0,)), ((), ())), precision=HI)
    rt = jnp.sum(x, axis=1, keepdims=True)
    r = lax.broadcasted_iota(I32, (NR, NR), 0)
    cc = lax.broadcasted_iota(I32, (NR, NR), 1)
    lower = jnp.where(cc < r, 1.0, 0.0)
    row_ex = lax.dot_general(lower, rt, (((1,), (0,)), ((), ())), precision=HI)
    return lane_ex + row_ex


def _pack_matrix():
    j = lax.broadcasted_iota(I32, (M, 64), 0)
    w = lax.broadcasted_iota(I32, (M, 64), 1)
    pw = jnp.left_shift(jnp.int32(1), jnp.bitwise_and(j, 15))
    return jnp.where(lax.shift_right_logical(j, 4) == w, pw, 0).astype(F32)


def _score_kernel(x_ref, v_ref, s_ref):
    scores = []
    for c in range(10):
        vt = jnp.transpose(x_ref[0, c * 2000:(c + 1) * 2000, :], (1, 0))
        obj = vt[4:5, :]
        p = vt[5:85, :] * obj
        conf = jnp.max(p, axis=0, keepdims=True)
        io = lax.broadcasted_iota(I32, (80, 2000), 0)
        cls = jnp.min(jnp.where(p == conf, io, 80), axis=0, keepdims=True)
        valid = (obj > CONF_T) & (conf > CONF_T)
        score = jnp.where(valid, conf, -1.0)
        hw = vt[2:3, :] * 0.5
        hh = vt[3:4, :] * 0.5
        out = jnp.concatenate(
            [vt[0:1, :] - hw, vt[1:2, :] - hh,
             vt[0:1, :] + hw, vt[1:2, :] + hh,
             score, cls.astype(F32), jnp.zeros((2, 2000), F32)], axis=0)
        v_ref[0, c * 2000:(c + 1) * 2000, :] = jnp.transpose(out, (1, 0))
        scores.append(score)
    scores.append(jnp.full((1, NP - N), -1.0, F32))
    v_ref[0, N:NP, :] = jnp.zeros((NP - N, 8), F32)
    s_ref[0] = jnp.concatenate(scores, axis=1).reshape(NR, 128)


_score_call = pl.pallas_call(
    _score_kernel,
    grid=(B,),
    in_specs=[pl.BlockSpec((1, N, 85), lambda i: (i, 0, 0))],
    out_specs=[pl.BlockSpec((1, NP, 8), lambda i: (i, 0, 0)),
               pl.BlockSpec((1, NR, 128), lambda i: (i, 0, 0))],
    out_shape=[SDS((B, NP, 8), F32), SDS((B, NR, 128), F32)],
)


def _thresh_kernel(s_ref, t_ref):
    keys = _f2key(s_ref[...])

    def body(bi, t):
        trial = t + jnp.left_shift(jnp.int32(1), 30 - bi)
        cnt = jnp.sum(jnp.where(keys >= trial, 1.0, 0.0), axis=(1, 2),
                      keepdims=True)
        return jnp.where(cnt >= float(M), trial, t)

    cnt0 = jnp.sum(jnp.where(keys >= 0, 1.0, 0.0), axis=(1, 2), keepdims=True)
    t0 = jnp.where(cnt0 >= float(M), jnp.int32(0),
                   jnp.full((B, 1, 1), jnp.iinfo(jnp.int32).min, I32))
    t = lax.fori_loop(0, 31, body, t0)
    c_above = jnp.sum(jnp.where(keys > t, 1.0, 0.0), axis=(1, 2),
                      keepdims=True).astype(I32)
    io = lax.broadcasted_iota(I32, (B, 1, 128), 2)
    t_ref[...] = jnp.where(io == 0, t, jnp.where(io == 1, c_above, 0))


_thresh_call = pl.pallas_call(
    _thresh_kernel,
    in_specs=[pl.BlockSpec((B, NR, 128), lambda: (0, 0, 0))],
    out_specs=pl.BlockSpec((B, 1, 128), lambda: (0, 0, 0)),
    out_shape=SDS((B, 1, 128), I32),
)


def _compact_kernel(s_ref, v_ref, t_ref, so_ref, st_ref, pos_s, sel_s):
    s = s_ref[0]
    keys = _f2key(s)
    t = t_ref[0]
    thr = t[0:1, 0:1]
    c_above = t[0:1, 1:2].astype(F32)
    m_gt = keys > thr
    m_eq = keys == thr
    ex_eq = _excl_cumsum(m_eq)
    n_need = float(M) - c_above
    sel = m_gt | (m_eq & (ex_eq < n_need))
    pos = _excl_cumsum(sel)
    pos_s[...] = pos.reshape(NCH, 1, CH)
    sel_s[...] = jnp.where(sel, 1.0, 0.0).reshape(NCH, 1, CH)

    iom = lax.broadcasted_iota(I32, (M, CH), 0).astype(F32)

    def chunk_body(c, acc):
        pc = pos_s[c]
        sc = sel_s[c]
        mc = jnp.where((iom == pc) & (sc > 0.5), 1.0, 0.0)
        vc = v_ref[0, pl.ds(c * CH, CH), :]
        return acc + lax.dot_general(mc, vc, (((1,), (0,)), ((), ())),
                                     precision=HI)

    acc = lax.fori_loop(0, NCH, chunk_body, jnp.zeros((M, 8), F32))

    acc_t = jnp.transpose(acc, (1, 0))
    kcol = _f2key(acc[:, 4:5])
    krow = _f2key(acc_t[4:5, :])
    icol = lax.broadcasted_iota(I32, (M, M), 0)
    irow = lax.broadcasted_iota(I32, (M, M), 1)
    beats = (krow > kcol) | ((krow == kcol) & (irow < icol))
    rank = lax.dot_general(jnp.where(beats, 1.0, 0.0), jnp.ones((M, 1), F32),
                           (((1,), (0,)), ((), ())), precision=HI)
    rank_row = jnp.transpose(rank, (1, 0))
    iop = lax.broadcasted_iota(I32, (M, M), 0).astype(F32)
    perm = jnp.where(iop == rank_row, 1.0, 0.0)
    srt = lax.dot_general(perm, acc, (((1,), (0,)), ((), ())), precision=HI)
    so_ref[0] = srt
    st_ref[0] = jnp.transpose(srt, (1, 0))


_compact_call = pl.pallas_call(
    _compact_kernel,
    grid=(B,),
    in_specs=[pl.BlockSpec((1, NR, 128), lambda i: (i, 0, 0)),
              pl.BlockSpec((1, NP, 8), lambda i: (i, 0, 0)),
              pl.BlockSpec((1, 1, 128), lambda i: (i, 0, 0))],
    out_specs=[pl.BlockSpec((1, M, 8), lambda i: (i, 0, 0)),
               pl.BlockSpec((1, 8, M), lambda i: (i, 0, 0))],
    out_shape=[SDS((B, M, 8), F32), SDS((B, 8, M), F32)],
    scratch_shapes=[pltpu.VMEM((NCH, 1, CH), F32),
                    pltpu.VMEM((NCH, 1, CH), F32)],
)


def _iou_kernel(so_ref, st_ref, ap_ref, vp_ref):
    bc = so_ref[0]
    br = st_ref[0]
    offc = bc[:, 5:6] * MAX_WH
    offr = br[5:6, :] * MAX_WH
    x1 = bc[:, 0:1] + offc
    y1 = bc[:, 1:2] + offc
    x2 = bc[:, 2:3] + offc
    y2 = bc[:, 3:4] + offc
    xx1 = br[0:1, :] + offr
    yy1 = br[1:2, :] + offr
    xx2 = br[2:3, :] + offr
    yy2 = br[3:4, :] + offr
    w = jnp.clip(jnp.minimum(x2, xx2) - jnp.maximum(x1, xx1), 0.0, None)
    h = jnp.clip(jnp.minimum(y2, yy2) - jnp.maximum(y1, yy1), 0.0, None)
    inter = w * h
    area = (x2 - x1) * (y2 - y1)
    area_t = (xx2 - xx1) * (yy2 - yy1)
    iou = inter / (area + area_t - inter + 1e-9)
    adj = jnp.where(iou > IOU_T, 1.0, 0.0)
    wp = _pack_matrix()
    ap_ref[0] = lax.dot_general(adj, wp, (((1,), (0,)), ((), ())),
                                precision=HI)
    vmask = jnp.where(br[4:5, :] > 0.0, 1.0, 0.0)
    vp_ref[0] = lax.dot_general(vmask, wp, (((1,), (0,)), ((), ())),
                                precision=HI)


_iou_call = pl.pallas_call(
    _iou_kernel,
    grid=(B,),
    in_specs=[pl.BlockSpec((1, M, 8), lambda i: (i, 0, 0)),
              pl.BlockSpec((1, 8, M), lambda i: (i, 0, 0))],
    out_specs=[pl.BlockSpec((1, M, 64), lambda i: (i, 0, 0)),
               pl.BlockSpec((1, 1, 64), lambda i: (i, 0, 0))],
    out_shape=[SDS((B, M, 64), F32), SDS((B, 1, 64), F32)],
)


def _greedy_kernel(ap_ref, vp_ref, keep_ref):
    validb = vp_ref[...][:, 0, :].astype(I32)
    io64 = lax.broadcasted_iota(I32, (1, 64), 1)
    io1k = lax.broadcasted_iota(I32, (B, M), 1)

    def body(i, carry):
        kbits, kf = carry
        row = ap_ref[:, pl.ds(i, 1), :][:, 0, :].astype(I32)
        supp = jnp.max(jnp.where((row & kbits) != 0, 1, 0), axis=1,
                       keepdims=True)
        word = lax.shift_right_logical(i, 4)
        bit = jnp.left_shift(jnp.int32(1), jnp.bitwise_and(i, 15))
        mvec = jnp.where(io64 == word, bit, 0)
        vbit = jnp.max(jnp.where((validb & mvec) != 0, 1, 0), axis=1,
                       keepdims=True)
        ok = (supp == 0) & (vbit == 1)
        kbits = kbits | jnp.where(ok, mvec, 0)
        kf = kf + jnp.where((io1k == i) & ok, 1.0, 0.0)
        return kbits, kf

    _, kf = lax.fori_loop(
        0, M, body,
        (jnp.zeros((B, 64), I32), jnp.zeros((B, M), F32)))
    keep_ref[...] = kf


_greedy_call = pl.pallas_call(
    _greedy_kernel,
    in_specs=[pl.BlockSpec((B, M, 64), lambda: (0, 0, 0)),
              pl.BlockSpec((B, 1, 64), lambda: (0, 0, 0))],
    out_specs=pl.BlockSpec((B, M), lambda: (0, 0)),
    out_shape=SDS((B, M), F32),
)


def _final_kernel(k_ref, so_ref, o_ref):
    i = pl.program_id(0)
    krow = k_ref[pl.ds(i, 1), :]
    kcol = jnp.transpose(krow, (1, 0))
    r = lax.broadcasted_iota(I32, (M, M), 0)
    c = lax.broadcasted_iota(I32, (M, M), 1)
    lower = jnp.where(c < r, 1.0, 0.0)
    pos = lax.dot_general(lower, kcol, (((1,), (0,)), ((), ())), precision=HI)
    pos_row = jnp.transpose(pos, (1, 0))
    iop = lax.broadcasted_iota(I32, (NPAD, M), 0).astype(F32)
    p2 = jnp.where((iop == pos_row) & (krow > 0.5), 1.0, 0.0)
    o_ref[0] = lax.dot_general(p2, so_ref[0], (((1,), (0,)), ((), ())),
                               precision=HI)


_final_call = pl.pallas_call(
    _final_kernel,
    grid=(B,),
    in_specs=[pl.BlockSpec((B, M), lambda i: (0, 0)),
              pl.BlockSpec((1, M, 8), lambda i: (i, 0, 0))],
    out_specs=pl.BlockSpec((1, NPAD, 8), lambda i: (i, 0, 0)),
    out_shape=SDS((B, NPAD, 8), F32),
)


def kernel(x):
    pred = x[0]
    v, s = _score_call(pred)
    tinfo = _thresh_call(s)
    srt, srt_t = _compact_call(s, v, tinfo)
    adjp, validp = _iou_call(srt, srt_t)
    keep = _greedy_call(adjp, validp)
    det = _final_call(keep, srt)
    return det[:, :NDET, :6]


# speedup vs baseline: 33.8343x; 1.1590x over previous
import jax
import jax.numpy as jnp
from jax import lax
from jax.experimental import pallas as pl
from jax.experimental.pallas import tpu as pltpu

CONF_T = 0.25
IOU_T = 0.45
MAX_WH = 4096.0
M = 1024
NDET = 300
NPAD = 320
N = 20000
NP = 20480
NR = 160
B = 8
CH = 2048
NCH = NP // CH
HI = lax.Precision.HIGHEST
F32 = jnp.float32
I32 = jnp.int32
SDS = jax.ShapeDtypeStruct
KEYNEG = -1065353217


def _f2key(s):
    bits = lax.bitcast_convert_type(s, I32)
    return jnp.where(bits >= 0, bits, jnp.bitwise_xor(bits, jnp.int32(0x7FFFFFFF)))


def _excl_cumsum(m):
    x = jnp.where(m, 1.0, 0.0)
    l = lax.broadcasted_iota(I32, (128, 128), 0)
    c = lax.broadcasted_iota(I32, (128, 128), 1)
    upper = jnp.where(l < c, 1.0, 0.0)
    lane_ex = lax.dot_general(x, upper, (((1,), (0,)), ((), ())), precision=HI)
    rt = jnp.sum(x, axis=1, keepdims=True)
    r = lax.broadcasted_iota(I32, (NR, NR), 0)
    cc = lax.broadcasted_iota(I32, (NR, NR), 1)
    lower = jnp.where(cc < r, 1.0, 0.0)
    row_ex = lax.dot_general(lower, rt, (((1,), (0,)), ((), ())), precision=HI)
    return lane_ex + row_ex


def _pack_matrix():
    j = lax.broadcasted_iota(I32, (M, 64), 0)
    w = lax.broadcasted_iota(I32, (M, 64), 1)
    pw = jnp.left_shift(jnp.int32(1), jnp.bitwise_and(j, 15))
    return jnp.where(lax.shift_right_logical(j, 4) == w, pw, 0).astype(F32)


def _score_kernel(x_ref, v_ref, s_ref):
    scores = []
    for c in range(10):
        vt = jnp.transpose(x_ref[0, 0, c * 2000:(c + 1) * 2000, :], (1, 0))
        obj = vt[4:5, :]
        p = vt[5:85, :] * obj
        conf = jnp.max(p, axis=0, keepdims=True)
        io = lax.broadcasted_iota(I32, (80, 2000), 0)
        cls = jnp.min(jnp.where(p == conf, io, 80), axis=0, keepdims=True)
        valid = (obj > CONF_T) & (conf > CONF_T)
        score = jnp.where(valid, conf, -1.0)
        hw = vt[2:3, :] * 0.5
        hh = vt[3:4, :] * 0.5
        out = jnp.concatenate(
            [vt[0:1, :] - hw, vt[1:2, :] - hh,
             vt[0:1, :] + hw, vt[1:2, :] + hh,
             score, cls.astype(F32), jnp.zeros((2, 2000), F32)], axis=0)
        v_ref[0, c * 2000:(c + 1) * 2000, :] = jnp.transpose(out, (1, 0))
        scores.append(score)
    scores.append(jnp.full((1, NP - N), -1.0, F32))
    v_ref[0, N:NP, :] = jnp.zeros((NP - N, 8), F32)
    s_ref[0] = jnp.concatenate(scores, axis=1).reshape(NR, 128)


_score_call = pl.pallas_call(
    _score_kernel,
    grid=(B,),
    in_specs=[pl.BlockSpec((1, 1, N, 85), lambda i: (0, i, 0, 0))],
    out_specs=[pl.BlockSpec((1, NP, 8), lambda i: (i, 0, 0)),
               pl.BlockSpec((1, NR, 128), lambda i: (i, 0, 0))],
    out_shape=[SDS((B, NP, 8), F32), SDS((B, NR, 128), F32)],
)


def _thresh_kernel(s_ref, t_ref):
    keys = _f2key(s_ref[...])

    def body(bi, t):
        trial = t + jnp.left_shift(jnp.int32(1), 30 - bi)
        cnt = jnp.sum(jnp.where(keys >= trial, 1.0, 0.0), axis=(1, 2),
                      keepdims=True)
        return jnp.where(cnt >= float(M), trial, t)

    cnt0 = jnp.sum(jnp.where(keys >= 0, 1.0, 0.0), axis=(1, 2), keepdims=True)
    t0 = jnp.where(cnt0 >= float(M), jnp.int32(0),
                   jnp.full((B, 1, 1), jnp.iinfo(jnp.int32).min, I32))
    t = lax.fori_loop(0, 31, body, t0)
    c_above = jnp.sum(jnp.where(keys > t, 1.0, 0.0), axis=(1, 2),
                      keepdims=True).astype(I32)
    io = lax.broadcasted_iota(I32, (B, 1, 128), 2)
    t_ref[...] = jnp.where(io == 0, t, jnp.where(io == 1, c_above, 0))


_thresh_call = pl.pallas_call(
    _thresh_kernel,
    in_specs=[pl.BlockSpec((B, NR, 128), lambda: (0, 0, 0))],
    out_specs=pl.BlockSpec((B, 1, 128), lambda: (0, 0, 0)),
    out_shape=SDS((B, 1, 128), I32),
)


def _compact_kernel(s_ref, v_ref, t_ref, so_ref, st_ref, pos_s, sel_s):
    s = s_ref[0]
    keys = _f2key(s)
    t = t_ref[0]
    thr = t[0:1, 0:1]
    c_above = t[0:1, 1:2].astype(F32)
    m_gt = keys > thr
    m_eq = keys == thr
    ex_eq = _excl_cumsum(m_eq)
    n_need = float(M) - c_above
    sel = m_gt | (m_eq & (ex_eq < n_need))
    pos = _excl_cumsum(sel)
    pos_s[...] = pos.reshape(NCH, 1, CH)
    sel_s[...] = jnp.where(sel, 1.0, 0.0).reshape(NCH, 1, CH)

    iom = lax.broadcasted_iota(I32, (M, CH), 0).astype(F32)

    def chunk_body(c, acc):
        pc = pos_s[c]
        sc = sel_s[c]
        mc = jnp.where((iom == pc) & (sc > 0.5), 1.0, 0.0)
        vc = v_ref[0, pl.ds(c * CH, CH), :]
        return acc + lax.dot_general(mc, vc, (((1,), (0,)), ((), ())),
                                     precision=HI)

    acc = lax.fori_loop(0, NCH, chunk_body, jnp.zeros((M, 8), F32))

    acc_t = jnp.transpose(acc, (1, 0))
    kcol = _f2key(acc[:, 4:5])
    krow = _f2key(acc_t[4:5, :])
    icol = lax.broadcasted_iota(I32, (M, M), 0)
    irow = lax.broadcasted_iota(I32, (M, M), 1)
    beats = (krow > kcol) | ((krow == kcol) & (irow < icol))
    rank = lax.dot_general(jnp.where(beats, 1.0, 0.0), jnp.ones((M, 1), F32),
                           (((1,), (0,)), ((), ())), precision=HI)
    rank_row = jnp.transpose(rank, (1, 0))
    iop = lax.broadcasted_iota(I32, (M, M), 0).astype(F32)
    perm = jnp.where(iop == rank_row, 1.0, 0.0)
    srt = lax.dot_general(perm, acc, (((1,), (0,)), ((), ())), precision=HI)
    so_ref[0] = srt
    st_ref[0] = jnp.transpose(srt, (1, 0))


_compact_call = pl.pallas_call(
    _compact_kernel,
    grid=(B,),
    in_specs=[pl.BlockSpec((1, NR, 128), lambda i: (i, 0, 0)),
              pl.BlockSpec((1, NP, 8), lambda i: (i, 0, 0)),
              pl.BlockSpec((1, 1, 128), lambda i: (i, 0, 0))],
    out_specs=[pl.BlockSpec((1, M, 8), lambda i: (i, 0, 0)),
               pl.BlockSpec((1, 8, M), lambda i: (i, 0, 0))],
    out_shape=[SDS((B, M, 8), F32), SDS((B, 8, M), F32)],
    scratch_shapes=[pltpu.VMEM((NCH, 1, CH), F32),
                    pltpu.VMEM((NCH, 1, CH), F32)],
)


def _iou_kernel(so_ref, st_ref, ap_ref, vp_ref):
    bc = so_ref[0]
    br = st_ref[0]
    offc = bc[:, 5:6] * MAX_WH
    offr = br[5:6, :] * MAX_WH
    x1 = bc[:, 0:1] + offc
    y1 = bc[:, 1:2] + offc
    x2 = bc[:, 2:3] + offc
    y2 = bc[:, 3:4] + offc
    xx1 = br[0:1, :] + offr
    yy1 = br[1:2, :] + offr
    xx2 = br[2:3, :] + offr
    yy2 = br[3:4, :] + offr
    w = jnp.clip(jnp.minimum(x2, xx2) - jnp.maximum(x1, xx1), 0.0, None)
    h = jnp.clip(jnp.minimum(y2, yy2) - jnp.maximum(y1, yy1), 0.0, None)
    inter = w * h
    area = (x2 - x1) * (y2 - y1)
    area_t = (xx2 - xx1) * (yy2 - yy1)
    iou = inter / (area + area_t - inter + 1e-9)
    adj = jnp.where(iou > IOU_T, 1.0, 0.0)
    wp = _pack_matrix()
    ap_ref[0] = lax.dot_general(adj, wp, (((1,), (0,)), ((), ())),
                                precision=HI)
    vmask = jnp.where(br[4:5, :] > 0.0, 1.0, 0.0)
    vp_ref[0] = lax.dot_general(vmask, wp, (((1,), (0,)), ((), ())),
                                precision=HI)


_iou_call = pl.pallas_call(
    _iou_kernel,
    grid=(B,),
    in_specs=[pl.BlockSpec((1, M, 8), lambda i: (i, 0, 0)),
              pl.BlockSpec((1, 8, M), lambda i: (i, 0, 0))],
    out_specs=[pl.BlockSpec((1, M, 64), lambda i: (i, 0, 0)),
               pl.BlockSpec((1, 1, 64), lambda i: (i, 0, 0))],
    out_shape=[SDS((B, M, 64), F32), SDS((B, 1, 64), F32)],
)


def _greedy_kernel(ap_ref, vp_ref, keep_ref):
    validb = vp_ref[...][:, 0, :].astype(I32)
    io64 = lax.broadcasted_iota(I32, (1, 64), 1)
    io1k = lax.broadcasted_iota(I32, (B, M), 1)

    def body(i, carry):
        kbits, kf = carry
        row = ap_ref[:, pl.ds(i, 1), :][:, 0, :].astype(I32)
        supp = jnp.max(jnp.where((row & kbits) != 0, 1, 0), axis=1,
                       keepdims=True)
        word = lax.shift_right_logical(i, 4)
        bit = jnp.left_shift(jnp.int32(1), jnp.bitwise_and(i, 15))
        mvec = jnp.where(io64 == word, bit, 0)
        vbit = jnp.max(jnp.where((validb & mvec) != 0, 1, 0), axis=1,
                       keepdims=True)
        ok = (supp == 0) & (vbit == 1)
        kbits = kbits | jnp.where(ok, mvec, 0)
        kf = kf + jnp.where((io1k == i) & ok, 1.0, 0.0)
        return kbits, kf

    _, kf = lax.fori_loop(
        0, M, body,
        (jnp.zeros((B, 64), I32), jnp.zeros((B, M), F32)))
    keep_ref[...] = kf


_greedy_call = pl.pallas_call(
    _greedy_kernel,
    in_specs=[pl.BlockSpec((B, M, 64), lambda: (0, 0, 0)),
              pl.BlockSpec((B, 1, 64), lambda: (0, 0, 0))],
    out_specs=pl.BlockSpec((B, M), lambda: (0, 0)),
    out_shape=SDS((B, M), F32),
)


def _final_kernel(k_ref, so_ref, o_ref):
    i = pl.program_id(0)
    krow = k_ref[pl.ds(i, 1), :]
    kcol = jnp.transpose(krow, (1, 0))
    r = lax.broadcasted_iota(I32, (M, M), 0)
    c = lax.broadcasted_iota(I32, (M, M), 1)
    lower = jnp.where(c < r, 1.0, 0.0)
    pos = lax.dot_general(lower, kcol, (((1,), (0,)), ((), ())), precision=HI)
    pos_row = jnp.transpose(pos, (1, 0))
    iop = lax.broadcasted_iota(I32, (NPAD, M), 0).astype(F32)
    p2 = jnp.where((iop == pos_row) & (krow > 0.5), 1.0, 0.0)
    o_ref[0] = lax.dot_general(p2, so_ref[0], (((1,), (0,)), ((), ())),
                               precision=HI)


_final_call = pl.pallas_call(
    _final_kernel,
    grid=(B,),
    in_specs=[pl.BlockSpec((B, M), lambda i: (0, 0)),
              pl.BlockSpec((1, M, 8), lambda i: (i, 0, 0))],
    out_specs=pl.BlockSpec((1, NPAD, 8), lambda i: (i, 0, 0)),
    out_shape=SDS((B, NPAD, 8), F32),
)


def kernel(x):
    v, s = _score_call(x)
    tinfo = _thresh_call(s)
    srt, srt_t = _compact_call(s, v, tinfo)
    adjp, validp = _iou_call(srt, srt_t)
    keep = _greedy_call(adjp, validp)
    det = _final_call(keep, srt)
    return det[:, :NDET, :6]
